# Initial kernel scaffold; baseline (speedup 1.0000x reference)
import jax
import jax.numpy as jnp
from jax.experimental import pallas as pl


def kernel(x, neigh, Ws, bs, gammas, betas, W_out, b_out):
    raise NotImplementedError("write your pallas kernel here")


# speedup vs baseline: 1.3444x; 1.3444x over previous
import functools

import jax
import jax.numpy as jnp
from jax import lax
from jax.experimental import pallas as pl
from jax.experimental.pallas import tpu as pltpu
from jax.experimental.pallas import tpu_sc as plsc

N = 10242
NP = 10496
K7 = 7
B = K7 * NP
NC, NS = 2, 16
NW = NC * NS
BPW = B // NW
CHUNK = 128
NFULL = BPW // CHUNK
REM = BPW - NFULL * CHUNK


def _make_gather(nrows):
    d = 128
    mesh = plsc.VectorSubcoreMesh(
        core_axis_name="c", subcore_axis_name="s",
        num_cores=NC, num_subcores=NS)

    @functools.partial(
        pl.kernel,
        mesh=mesh,
        out_type=jax.ShapeDtypeStruct((B, d), jnp.float32),
        scratch_types=[
            pltpu.VMEM((BPW,), jnp.int32),
            pltpu.VMEM((CHUNK, d), jnp.float32),
            pltpu.SemaphoreType.DMA,
        ],
    )
    def gather_kernel(table_hbm, idx_hbm, out_hbm, idx_v, rows_v, gsem):
        wid = lax.axis_index("s") * NC + lax.axis_index("c")
        base = wid * BPW
        pltpu.sync_copy(idx_hbm.at[pl.ds(base, BPW)], idx_v)
        for c in range(NFULL + 1):
            sz = CHUNK if c < NFULL else REM
            off = c * CHUNK
            pltpu.async_copy(
                table_hbm.at[idx_v.at[pl.ds(off, sz)]],
                rows_v.at[pl.ds(0, sz)], gsem).wait()
            pltpu.sync_copy(rows_v.at[pl.ds(0, sz)],
                            out_hbm.at[pl.ds(base + off, sz)])

    return gather_kernel


_gather_cache = {}


def _gather(table, idx):
    key = table.shape
    if key not in _gather_cache:
        _gather_cache[key] = _make_gather(table.shape[0])
    return _gather_cache[key](table, idx)


def _col_sum(z):
    acc = z[0:128]
    for g in range(1, NP // 128):
        acc = acc + z[g * 128:(g + 1) * 128]
    a = acc[0:8]
    for k in range(1, 16):
        a = a + acc[k * 8:(k + 1) * 8]
    return jnp.sum(a, axis=0, keepdims=True)


def _cat_dot(g_ref, w_ref, cin):
    blk = 656
    parts = []
    for r in range(0, NP, blk):
        cat = jnp.concatenate(
            [g_ref[j][r:r + blk, :cin] for j in range(K7)], axis=1)
        parts.append(jnp.dot(cat, w_ref[...],
                             preferred_element_type=jnp.float32))
    return jnp.concatenate(parts, axis=0)


def _layer_body(g_ref, w_ref, b_ref, gm_ref, bt_ref, o_ref, *, cin):
    m = _cat_dot(g_ref, w_ref, cin) + b_ref[...]
    mask = (lax.broadcasted_iota(jnp.int32, (NP, 1), 0) < N).astype(jnp.float32)
    mean = _col_sum(m * mask) / float(N)
    ctr = m - mean
    var = _col_sum(ctr * ctr * mask) / float(N)
    hn = (ctr / jnp.sqrt(var + 1e-5)) * gm_ref[...] + bt_ref[...]
    act = jnp.maximum(hn, 0.0) * mask
    o_ref[...] = jnp.concatenate([act, jnp.zeros_like(act)], axis=1)


def _final_body(g_ref, w_ref, b_ref, o_ref):
    o_ref[...] = _cat_dot(g_ref, w_ref, 64) + b_ref[...]


def _tc_layer(g3, w, b, gamma, beta, cin):
    return pl.pallas_call(
        functools.partial(_layer_body, cin=cin),
        out_shape=jax.ShapeDtypeStruct((NP, 128), jnp.float32),
    )(g3, w, b.reshape(1, -1), gamma.reshape(1, -1), beta.reshape(1, -1))


def _tc_final(g3, w, b):
    return pl.pallas_call(
        _final_body,
        out_shape=jax.ShapeDtypeStruct((NP, w.shape[1]), jnp.float32),
    )(g3, w, b.reshape(1, -1))


def kernel(x, neigh, Ws, bs, gammas, betas, W_out, b_out):
    ni = neigh.reshape(N, K7).T
    ni = jnp.pad(ni, ((0, 0), (0, NP - N))).reshape(-1)

    xp = jnp.pad(x, ((0, 0), (0, 124)))
    g3 = _gather(xp, ni).reshape(K7, NP, 128)
    h = _tc_layer(g3, Ws[0], bs[0], gammas[0], betas[0], cin=4)

    for l in range(1, len(Ws)):
        g3 = _gather(h, ni).reshape(K7, NP, 128)
        h = _tc_layer(g3, Ws[l], bs[l], gammas[l], betas[l], cin=64)

    g3 = _gather(h, ni).reshape(K7, NP, 128)
    out = _tc_final(g3, W_out, b_out)
    return out[:N]


# speedup vs baseline: 1.4337x; 1.0665x over previous
import functools

import jax
import jax.numpy as jnp
from jax import lax
from jax.experimental import pallas as pl
from jax.experimental.pallas import tpu as pltpu
from jax.experimental.pallas import tpu_sc as plsc

N = 10242
NP = 10496
K7 = 7
B = K7 * NP
NC, NS = 2, 16
NW = NC * NS
BPW = B // NW
CHUNK = 128
NFULL = BPW // CHUNK
REM = BPW - NFULL * CHUNK


NBUF = 4


def _make_gather(nrows):
    d = 128
    mesh = plsc.VectorSubcoreMesh(
        core_axis_name="c", subcore_axis_name="s",
        num_cores=NC, num_subcores=NS)

    @functools.partial(
        pl.kernel,
        mesh=mesh,
        out_type=jax.ShapeDtypeStruct((B, d), jnp.float32),
        scratch_types=[
            pltpu.VMEM((BPW,), jnp.int32),
            pltpu.VMEM((NBUF, CHUNK, d), jnp.float32),
            [pltpu.SemaphoreType.DMA] * NBUF,
            [pltpu.SemaphoreType.DMA] * NBUF,
        ],
    )
    def gather_kernel(table_hbm, idx_hbm, out_hbm, idx_v, rows_v, gsems, osems):
        wid = lax.axis_index("s") * NC + lax.axis_index("c")
        base = wid * BPW
        pltpu.sync_copy(idx_hbm.at[pl.ds(base, BPW)], idx_v)
        nch = NFULL + 1
        gd = [None] * nch
        od = [None] * nch
        for t in range(nch + 1):
            if t < nch:
                sl = t % NBUF
                if t >= NBUF:
                    od[t - NBUF].wait()
                sz = CHUNK if t < NFULL else REM
                gd[t] = pltpu.async_copy(
                    table_hbm.at[idx_v.at[pl.ds(t * CHUNK, sz)]],
                    rows_v.at[sl, pl.ds(0, sz)], gsems[sl])
            if t >= 1:
                c = t - 1
                sl = c % NBUF
                sz = CHUNK if c < NFULL else REM
                gd[c].wait()
                od[c] = pltpu.async_copy(
                    rows_v.at[sl, pl.ds(0, sz)],
                    out_hbm.at[pl.ds(base + c * CHUNK, sz)], osems[sl])
        for c in range(nch - NBUF, nch):
            if c >= 0:
                od[c].wait()

    return gather_kernel


_gather_cache = {}


def _gather(table, idx):
    key = table.shape
    if key not in _gather_cache:
        _gather_cache[key] = _make_gather(table.shape[0])
    return _gather_cache[key](table, idx)


def _col_sum(z):
    acc = z[0:128]
    for g in range(1, NP // 128):
        acc = acc + z[g * 128:(g + 1) * 128]
    a = acc[0:8]
    for k in range(1, 16):
        a = a + acc[k * 8:(k + 1) * 8]
    return jnp.sum(a, axis=0, keepdims=True)


def _cat_dot(g_ref, w_ref, cin):
    blk = 656
    parts = []
    for r in range(0, NP, blk):
        cat = jnp.concatenate(
            [g_ref[j][r:r + blk, :cin] for j in range(K7)], axis=1)
        parts.append(jnp.dot(cat, w_ref[...],
                             preferred_element_type=jnp.float32))
    return jnp.concatenate(parts, axis=0)


def _layer_body(g_ref, w_ref, b_ref, gm_ref, bt_ref, o_ref, *, cin):
    m = _cat_dot(g_ref, w_ref, cin) + b_ref[...]
    mask = (lax.broadcasted_iota(jnp.int32, (NP, 1), 0) < N).astype(jnp.float32)
    mean = _col_sum(m * mask) / float(N)
    ctr = m - mean
    var = _col_sum(ctr * ctr * mask) / float(N)
    hn = (ctr / jnp.sqrt(var + 1e-5)) * gm_ref[...] + bt_ref[...]
    act = jnp.maximum(hn, 0.0) * mask
    o_ref[...] = jnp.concatenate([act, jnp.zeros_like(act)], axis=1)


def _final_body(g_ref, w_ref, b_ref, o_ref):
    o_ref[...] = _cat_dot(g_ref, w_ref, 64) + b_ref[...]


def _tc_layer(g3, w, b, gamma, beta, cin):
    return pl.pallas_call(
        functools.partial(_layer_body, cin=cin),
        out_shape=jax.ShapeDtypeStruct((NP, 128), jnp.float32),
    )(g3, w, b.reshape(1, -1), gamma.reshape(1, -1), beta.reshape(1, -1))


def _tc_final(g3, w, b):
    return pl.pallas_call(
        _final_body,
        out_shape=jax.ShapeDtypeStruct((NP, w.shape[1]), jnp.float32),
    )(g3, w, b.reshape(1, -1))


def kernel(x, neigh, Ws, bs, gammas, betas, W_out, b_out):
    ni = neigh.reshape(N, K7).T
    ni = jnp.pad(ni, ((0, 0), (0, NP - N))).reshape(-1)

    xp = jnp.pad(x, ((0, 0), (0, 124)))
    g3 = _gather(xp, ni).reshape(K7, NP, 128)
    h = _tc_layer(g3, Ws[0], bs[0], gammas[0], betas[0], cin=4)

    for l in range(1, len(Ws)):
        g3 = _gather(h, ni).reshape(K7, NP, 128)
        h = _tc_layer(g3, Ws[l], bs[l], gammas[l], betas[l], cin=64)

    g3 = _gather(h, ni).reshape(K7, NP, 128)
    out = _tc_final(g3, W_out, b_out)
    return out[:N]


# speedup vs baseline: 1.4652x; 1.0220x over previous
import functools

import jax
import jax.numpy as jnp
from jax import lax
from jax.experimental import pallas as pl
from jax.experimental.pallas import tpu as pltpu
from jax.experimental.pallas import tpu_sc as plsc

N = 10242
NP = 10496
K7 = 7
B = K7 * NP
NC, NS = 2, 16
NW = NC * NS
BPW = B // NW
CHUNK = 328
NFULL = BPW // CHUNK
REM = BPW - NFULL * CHUNK


NBUF = 2


def _make_gather(nrows):
    d = 128
    mesh = plsc.VectorSubcoreMesh(
        core_axis_name="c", subcore_axis_name="s",
        num_cores=NC, num_subcores=NS)

    @functools.partial(
        pl.kernel,
        mesh=mesh,
        out_type=jax.ShapeDtypeStruct((B, d), jnp.float32),
        scratch_types=[
            pltpu.VMEM((BPW,), jnp.int32),
            pltpu.VMEM((NBUF, CHUNK, d), jnp.float32),
            [pltpu.SemaphoreType.DMA] * NBUF,
            [pltpu.SemaphoreType.DMA] * NBUF,
        ],
    )
    def gather_kernel(table_hbm, idx_hbm, out_hbm, idx_v, rows_v, gsems, osems):
        wid = lax.axis_index("s") * NC + lax.axis_index("c")
        base = wid * BPW
        pltpu.sync_copy(idx_hbm.at[pl.ds(base, BPW)], idx_v)
        nch = NFULL
        gd = [None] * nch
        od = [None] * nch
        for t in range(nch + 1):
            if t < nch:
                sl = t % NBUF
                if t >= NBUF:
                    od[t - NBUF].wait()
                sz = CHUNK
                gd[t] = pltpu.async_copy(
                    table_hbm.at[idx_v.at[pl.ds(t * CHUNK, sz)]],
                    rows_v.at[sl, pl.ds(0, sz)], gsems[sl])
            if t >= 1:
                c = t - 1
                sl = c % NBUF
                sz = CHUNK
                gd[c].wait()
                od[c] = pltpu.async_copy(
                    rows_v.at[sl, pl.ds(0, sz)],
                    out_hbm.at[pl.ds(base + c * CHUNK, sz)], osems[sl])
        for c in range(nch - NBUF, nch):
            if c >= 0:
                od[c].wait()

    return gather_kernel


_gather_cache = {}


def _gather(table, idx):
    key = table.shape
    if key not in _gather_cache:
        _gather_cache[key] = _make_gather(table.shape[0])
    return _gather_cache[key](table, idx)


def _col_sum(z):
    acc = z[0:128]
    for g in range(1, NP // 128):
        acc = acc + z[g * 128:(g + 1) * 128]
    a = acc[0:8]
    for k in range(1, 16):
        a = a + acc[k * 8:(k + 1) * 8]
    return jnp.sum(a, axis=0, keepdims=True)


def _cat_dot(g_ref, w_ref, cin):
    blk = 656
    parts = []
    for r in range(0, NP, blk):
        cat = jnp.concatenate(
            [g_ref[j][r:r + blk, :cin] for j in range(K7)], axis=1)
        parts.append(jnp.dot(cat, w_ref[...],
                             preferred_element_type=jnp.float32))
    return jnp.concatenate(parts, axis=0)


def _layer_body(g_ref, w_ref, b_ref, gm_ref, bt_ref, o_ref, *, cin):
    m = _cat_dot(g_ref, w_ref, cin) + b_ref[...]
    mask = (lax.broadcasted_iota(jnp.int32, (NP, 1), 0) < N).astype(jnp.float32)
    mean = _col_sum(m * mask) / float(N)
    ctr = m - mean
    var = _col_sum(ctr * ctr * mask) / float(N)
    hn = (ctr / jnp.sqrt(var + 1e-5)) * gm_ref[...] + bt_ref[...]
    act = jnp.maximum(hn, 0.0) * mask
    o_ref[...] = jnp.concatenate([act, jnp.zeros_like(act)], axis=1)


def _final_body(g_ref, w_ref, b_ref, o_ref):
    o_ref[...] = _cat_dot(g_ref, w_ref, 64) + b_ref[...]


def _tc_layer(g3, w, b, gamma, beta, cin):
    return pl.pallas_call(
        functools.partial(_layer_body, cin=cin),
        out_shape=jax.ShapeDtypeStruct((NP, 128), jnp.float32),
    )(g3, w, b.reshape(1, -1), gamma.reshape(1, -1), beta.reshape(1, -1))


def _tc_final(g3, w, b):
    return pl.pallas_call(
        _final_body,
        out_shape=jax.ShapeDtypeStruct((NP, w.shape[1]), jnp.float32),
    )(g3, w, b.reshape(1, -1))


def kernel(x, neigh, Ws, bs, gammas, betas, W_out, b_out):
    ni = neigh.reshape(N, K7).T
    ni = jnp.pad(ni, ((0, 0), (0, NP - N))).reshape(-1)

    xp = jnp.pad(x, ((0, 0), (0, 124)))
    g3 = _gather(xp, ni).reshape(K7, NP, 128)
    h = _tc_layer(g3, Ws[0], bs[0], gammas[0], betas[0], cin=4)

    for l in range(1, len(Ws)):
        g3 = _gather(h, ni).reshape(K7, NP, 128)
        h = _tc_layer(g3, Ws[l], bs[l], gammas[l], betas[l], cin=64)

    g3 = _gather(h, ni).reshape(K7, NP, 128)
    out = _tc_final(g3, W_out, b_out)
    return out[:N]


# speedup vs baseline: 1.7674x; 1.2063x over previous
import functools

import jax
import jax.numpy as jnp
from jax import lax
from jax.experimental import pallas as pl
from jax.experimental.pallas import tpu as pltpu
from jax.experimental.pallas import tpu_sc as plsc

N = 10242
NP = 10496
K7 = 7
B = K7 * NP
NC, NS = 2, 16
NW = NC * NS
BPW = B // NW
CHUNK = 328
NFULL = BPW // CHUNK
REM = BPW - NFULL * CHUNK


NBUF = 2


def _make_gather(nrows):
    d = 64
    mesh = plsc.VectorSubcoreMesh(
        core_axis_name="c", subcore_axis_name="s",
        num_cores=NC, num_subcores=NS)

    @functools.partial(
        pl.kernel,
        mesh=mesh,
        out_type=jax.ShapeDtypeStruct((B, d), jnp.float32),
        compiler_params=pltpu.CompilerParams(use_tc_tiling_on_sc=False),
        scratch_types=[
            pltpu.VMEM((BPW,), jnp.int32),
            pltpu.VMEM((NBUF, CHUNK, d), jnp.float32),
            [pltpu.SemaphoreType.DMA] * NBUF,
            [pltpu.SemaphoreType.DMA] * NBUF,
        ],
    )
    def gather_kernel(table_hbm, idx_hbm, out_hbm, idx_v, rows_v, gsems, osems):
        wid = lax.axis_index("s") * NC + lax.axis_index("c")
        base = wid * BPW
        pltpu.sync_copy(idx_hbm.at[pl.ds(base, BPW)], idx_v)
        nch = NFULL
        gd = [None] * nch
        od = [None] * nch
        for t in range(nch + 1):
            if t < nch:
                sl = t % NBUF
                if t >= NBUF:
                    od[t - NBUF].wait()
                sz = CHUNK
                gd[t] = pltpu.async_copy(
                    table_hbm.at[idx_v.at[pl.ds(t * CHUNK, sz)]],
                    rows_v.at[sl, pl.ds(0, sz)], gsems[sl])
            if t >= 1:
                c = t - 1
                sl = c % NBUF
                sz = CHUNK
                gd[c].wait()
                od[c] = pltpu.async_copy(
                    rows_v.at[sl, pl.ds(0, sz)],
                    out_hbm.at[pl.ds(base + c * CHUNK, sz)], osems[sl])
        for c in range(nch - NBUF, nch):
            if c >= 0:
                od[c].wait()

    return gather_kernel


_gather_cache = {}


def _gather(table, idx):
    key = table.shape
    if key not in _gather_cache:
        _gather_cache[key] = _make_gather(table.shape[0])
    return _gather_cache[key](table, idx)


def _col_sum(z):
    acc = z[0:128]
    for g in range(1, NP // 128):
        acc = acc + z[g * 128:(g + 1) * 128]
    a = acc[0:8]
    for k in range(1, 16):
        a = a + acc[k * 8:(k + 1) * 8]
    return jnp.sum(a, axis=0, keepdims=True)


def _cat_dot(g_ref, w_ref, cin):
    blk = 656
    parts = []
    for r in range(0, NP, blk):
        cat = jnp.concatenate(
            [g_ref[j][r:r + blk, :cin] for j in range(K7)], axis=1)
        parts.append(jnp.dot(cat, w_ref[...],
                             preferred_element_type=jnp.float32))
    return jnp.concatenate(parts, axis=0)


def _layer_body(g_ref, w_ref, b_ref, gm_ref, bt_ref, o_ref, *, cin):
    m = _cat_dot(g_ref, w_ref, cin) + b_ref[...]
    mask = (lax.broadcasted_iota(jnp.int32, (NP, 1), 0) < N).astype(jnp.float32)
    mean = _col_sum(m * mask) / float(N)
    ctr = m - mean
    var = _col_sum(ctr * ctr * mask) / float(N)
    hn = (ctr / jnp.sqrt(var + 1e-5)) * gm_ref[...] + bt_ref[...]
    o_ref[...] = jnp.maximum(hn, 0.0) * mask


def _final_body(g_ref, w_ref, b_ref, o_ref):
    o_ref[...] = _cat_dot(g_ref, w_ref, 64) + b_ref[...]


def _tc_layer(g3, w, b, gamma, beta, cin):
    return pl.pallas_call(
        functools.partial(_layer_body, cin=cin),
        out_shape=jax.ShapeDtypeStruct((NP, 64), jnp.float32),
    )(g3, w, b.reshape(1, -1), gamma.reshape(1, -1), beta.reshape(1, -1))


def _tc_final(g3, w, b):
    return pl.pallas_call(
        _final_body,
        out_shape=jax.ShapeDtypeStruct((NP, w.shape[1]), jnp.float32),
    )(g3, w, b.reshape(1, -1))


def kernel(x, neigh, Ws, bs, gammas, betas, W_out, b_out):
    ni = neigh.reshape(N, K7).T
    ni = jnp.pad(ni, ((0, 0), (0, NP - N))).reshape(-1)

    xp = jnp.pad(x, ((0, 0), (0, 60)))
    g3 = _gather(xp, ni).reshape(K7, NP, 64)
    h = _tc_layer(g3, Ws[0], bs[0], gammas[0], betas[0], cin=4)

    for l in range(1, len(Ws)):
        g3 = _gather(h, ni).reshape(K7, NP, 64)
        h = _tc_layer(g3, Ws[l], bs[l], gammas[l], betas[l], cin=64)

    g3 = _gather(h, ni).reshape(K7, NP, 64)
    out = _tc_final(g3, W_out, b_out)
    return out[:N]
